# Initial kernel scaffold; baseline (speedup 1.0000x reference)
import functools

import jax
import jax.numpy as jnp
from jax import lax
from jax.experimental import pallas as pl
from jax.experimental.pallas import tpu as pltpu
from jax.experimental.pallas import tpu_sc as plsc

_RANK = 12
_RES = 256
_OUT = 32
_L = 16
_B = 128
_NW = 32


def _sc_sample(np_pad, iters):
  mesh = plsc.VectorSubcoreMesh(core_axis_name="c", subcore_axis_name="s")
  chunk = np_pad // _NW

  scratch = (
      [pltpu.VMEM((_B,), jnp.float32) for _ in range(3)]
      + [pltpu.VMEM((_B,), jnp.float32) for _ in range(3)]
      + [pltpu.VMEM((_B,), jnp.int32) for _ in range(12)]
      + [pltpu.VMEM((_B, _L), jnp.float32) for _ in range(12)]
      + [pltpu.VMEM((_B, 3 * _L), jnp.float32)]
      + [pltpu.SemaphoreType.DMA]
  )

  @functools.partial(
      pl.kernel,
      mesh=mesh,
      out_type=jax.ShapeDtypeStruct((np_pad, 3 * _L), jnp.float32),
      scratch_types=scratch,
  )
  def k(txy, txz, tyz, xs, ys, zs, f_out,
        cx, cy, cz, wtx, wty, wtz,
        i0, i1, i2, i3, i4, i5, i6, i7, i8, i9, i10, i11,
        g0, g1, g2, g3, g4, g5, g6, g7, g8, g9, g10, g11,
        fbuf, sem):
    wid = lax.axis_index("s") * 2 + lax.axis_index("c")
    base = wid * chunk
    idx_refs = (i0, i1, i2, i3, i4, i5, i6, i7, i8, i9, i10, i11)
    g_refs = (g0, g1, g2, g3, g4, g5, g6, g7, g8, g9, g10, g11)

    def block(it, carry):
      off = base + it * _B
      pltpu.sync_copy(xs.at[pl.ds(off, _B)], cx)
      pltpu.sync_copy(ys.at[pl.ds(off, _B)], cy)
      pltpu.sync_copy(zs.at[pl.ds(off, _B)], cz)

      for g in range(_B // _L):
        sl = pl.ds(g * _L, _L)

        def coord(ref):
          w = ref[sl] * 255.0
          ui = jnp.clip(w.astype(jnp.int32), 0, 254)
          tu = jnp.clip(w - ui.astype(jnp.float32), 0.0, 1.0)
          return ui, tu

        xi, tx = coord(cx)
        yi, ty = coord(cy)
        zi, tz = coord(cz)
        wtx[sl] = tx
        wty[sl] = ty
        wtz[sl] = tz
        for p, (ci, ri) in enumerate(((xi, yi), (xi, zi), (yi, zi))):
          b00 = ri * 256 + ci
          idx_refs[4 * p + 0][sl] = b00
          idx_refs[4 * p + 1][sl] = b00 + 1
          idx_refs[4 * p + 2][sl] = b00 + 256
          idx_refs[4 * p + 3][sl] = b00 + 257

      copies = []
      for p, tab in enumerate((txy, txz, tyz)):
        for t in range(4):
          copies.append(
              pltpu.async_copy(tab.at[idx_refs[4 * p + t]], g_refs[4 * p + t],
                               sem))
      for cp in copies:
        cp.wait()

      def combine(p, c):
        tx = wtx[p]
        ty = wty[p]
        tz = wtz[p]
        for pi, (tc, tr) in enumerate(((tx, ty), (tx, tz), (ty, tz))):
          a = g_refs[4 * pi + 0][p]
          b = g_refs[4 * pi + 1][p]
          c2 = g_refs[4 * pi + 2][p]
          d = g_refs[4 * pi + 3][p]
          lo = a + tc * (b - a)
          hi = c2 + tc * (d - c2)
          fbuf[p, pl.ds(pi * _L, _L)] = lo + tr * (hi - lo)
        return c

      lax.fori_loop(0, _B, combine, 0)
      pltpu.sync_copy(fbuf, f_out.at[pl.ds(off, _B)])
      return carry

    lax.fori_loop(0, iters, block, 0)

  return k


def _tc_project(f, wp, np_pad):
  blk = 4096

  def mm(f_ref, w_ref, o_ref):
    o_ref[...] = jnp.dot(f_ref[...], w_ref[...],
                         preferred_element_type=jnp.float32)

  return pl.pallas_call(
      mm,
      grid=(np_pad // blk,),
      in_specs=[
          pl.BlockSpec((blk, 3 * _L), lambda i: (i, 0)),
          pl.BlockSpec((3 * _L, _OUT), lambda i: (0, 0)),
      ],
      out_specs=pl.BlockSpec((blk, _OUT), lambda i: (i, 0)),
      out_shape=jax.ShapeDtypeStruct((np_pad, _OUT), jnp.float32),
  )(f, wp)


def kernel(xyz, xy, xz, yz, W):
  n = xyz.shape[0]
  grain = _NW * _B
  np_pad = ((n + grain - 1) // grain) * grain
  iters = np_pad // (grain)

  def tab(p):
    return jnp.pad(p.transpose(1, 2, 0).reshape(_RES * _RES, _RANK),
                   ((0, 0), (0, _L - _RANK)))

  txy, txz, tyz = tab(xy), tab(xz), tab(yz)
  pad = np_pad - n
  xs = jnp.pad(xyz[:, 0], (0, pad))
  ys = jnp.pad(xyz[:, 1], (0, pad))
  zs = jnp.pad(xyz[:, 2], (0, pad))

  f = _sc_sample(np_pad, iters)(txy, txz, tyz, xs, ys, zs)

  wt = W.T
  wp = jnp.zeros((3 * _L, _OUT), jnp.float32)
  wp = (wp.at[0:_RANK].set(wt[0:_RANK])
          .at[_L:_L + _RANK].set(wt[_RANK:2 * _RANK])
          .at[2 * _L:2 * _L + _RANK].set(wt[2 * _RANK:3 * _RANK]))

  out = _tc_project(f, wp, np_pad)
  return out[:n]


# speedup vs baseline: 37.3229x; 37.3229x over previous
import functools

import jax
import jax.numpy as jnp
from jax import lax
from jax.experimental import pallas as pl
from jax.experimental.pallas import tpu as pltpu
from jax.experimental.pallas import tpu_sc as plsc

_RANK = 12
_RES = 256
_OUT = 32
_L = 16
_B = 128
_NW = 32


def _sc_sample(np_pad, iters):
  mesh = plsc.VectorSubcoreMesh(core_axis_name="c", subcore_axis_name="s")
  chunk = np_pad // _NW

  scratch = (
      [pltpu.VMEM((_B,), jnp.float32) for _ in range(3)]
      + [pltpu.VMEM((_B,), jnp.float32) for _ in range(3)]
      + [pltpu.VMEM((_B,), jnp.int32) for _ in range(12)]
      + [pltpu.VMEM((_B, _L), jnp.float32) for _ in range(12)]
      + [pltpu.VMEM((_B, 3 * _L), jnp.float32)]
      + [pltpu.SemaphoreType.DMA]
  )

  @functools.partial(
      pl.kernel,
      mesh=mesh,
      out_type=jax.ShapeDtypeStruct((np_pad, 3 * _L), jnp.float32),
      scratch_types=scratch,
      compiler_params=pltpu.CompilerParams(use_tc_tiling_on_sc=False),
  )
  def k(txy, txz, tyz, xs, ys, zs, f_out,
        cx, cy, cz, wtx, wty, wtz,
        i0, i1, i2, i3, i4, i5, i6, i7, i8, i9, i10, i11,
        g0, g1, g2, g3, g4, g5, g6, g7, g8, g9, g10, g11,
        fbuf, sem):
    wid = lax.axis_index("s") * 2 + lax.axis_index("c")
    base = wid * chunk
    idx_refs = (i0, i1, i2, i3, i4, i5, i6, i7, i8, i9, i10, i11)
    g_refs = (g0, g1, g2, g3, g4, g5, g6, g7, g8, g9, g10, g11)

    def block(it, carry):
      off = base + it * _B
      pltpu.sync_copy(xs.at[pl.ds(off, _B)], cx)
      pltpu.sync_copy(ys.at[pl.ds(off, _B)], cy)
      pltpu.sync_copy(zs.at[pl.ds(off, _B)], cz)

      for g in range(_B // _L):
        sl = pl.ds(g * _L, _L)

        def coord(ref):
          w = ref[sl] * 255.0
          ui = jnp.clip(w.astype(jnp.int32), 0, 254)
          tu = jnp.clip(w - ui.astype(jnp.float32), 0.0, 1.0)
          return ui, tu

        xi, tx = coord(cx)
        yi, ty = coord(cy)
        zi, tz = coord(cz)
        wtx[sl] = tx
        wty[sl] = ty
        wtz[sl] = tz
        for p, (ci, ri) in enumerate(((xi, yi), (xi, zi), (yi, zi))):
          b00 = ri * 256 + ci
          idx_refs[4 * p + 0][sl] = b00
          idx_refs[4 * p + 1][sl] = b00 + 1
          idx_refs[4 * p + 2][sl] = b00 + 256
          idx_refs[4 * p + 3][sl] = b00 + 257

      copies = []
      for p, tab in enumerate((txy, txz, tyz)):
        for t in range(4):
          copies.append(
              pltpu.async_copy(tab.at[idx_refs[4 * p + t]], g_refs[4 * p + t],
                               sem))
      for cp in copies:
        cp.wait()

      def combine(grp, c):
        base16 = grp * _L
        txv = wtx[pl.ds(base16, _L)]
        tyv = wty[pl.ds(base16, _L)]
        tzv = wtz[pl.ds(base16, _L)]
        for j in range(_L):
          p = base16 + j
          tx = txv[j]
          ty = tyv[j]
          tz = tzv[j]
          for pi, (tc, tr) in enumerate(((tx, ty), (tx, tz), (ty, tz))):
            a = g_refs[4 * pi + 0][p]
            b = g_refs[4 * pi + 1][p]
            c2 = g_refs[4 * pi + 2][p]
            d = g_refs[4 * pi + 3][p]
            lo = a + tc * (b - a)
            hi = c2 + tc * (d - c2)
            fbuf[p, pl.ds(pi * _L, _L)] = lo + tr * (hi - lo)
        return c

      lax.fori_loop(0, _B // _L, combine, 0)
      pltpu.sync_copy(fbuf, f_out.at[pl.ds(off, _B)])
      return carry

    lax.fori_loop(0, iters, block, 0)

  return k


def _tc_project(f, wp, np_pad):
  blk = 4096

  def mm(f_ref, w_ref, o_ref):
    o_ref[...] = jnp.dot(f_ref[...], w_ref[...],
                         preferred_element_type=jnp.float32)

  return pl.pallas_call(
      mm,
      grid=(np_pad // blk,),
      in_specs=[
          pl.BlockSpec((blk, 3 * _L), lambda i: (i, 0)),
          pl.BlockSpec((3 * _L, _OUT), lambda i: (0, 0)),
      ],
      out_specs=pl.BlockSpec((blk, _OUT), lambda i: (i, 0)),
      out_shape=jax.ShapeDtypeStruct((np_pad, _OUT), jnp.float32),
  )(f, wp)


def kernel(xyz, xy, xz, yz, W):
  n = xyz.shape[0]
  grain = _NW * _B
  np_pad = ((n + grain - 1) // grain) * grain
  iters = np_pad // (grain)

  def tab(p):
    return jnp.pad(p.transpose(1, 2, 0).reshape(_RES * _RES, _RANK),
                   ((0, 0), (0, _L - _RANK)))

  txy, txz, tyz = tab(xy), tab(xz), tab(yz)
  pad = np_pad - n
  xs = jnp.pad(xyz[:, 0], (0, pad))
  ys = jnp.pad(xyz[:, 1], (0, pad))
  zs = jnp.pad(xyz[:, 2], (0, pad))

  f = _sc_sample(np_pad, iters)(txy, txz, tyz, xs, ys, zs)

  wt = W.T
  wp = jnp.zeros((3 * _L, _OUT), jnp.float32)
  wp = (wp.at[0:_RANK].set(wt[0:_RANK])
          .at[_L:_L + _RANK].set(wt[_RANK:2 * _RANK])
          .at[2 * _L:2 * _L + _RANK].set(wt[2 * _RANK:3 * _RANK]))

  out = _tc_project(f, wp, np_pad)
  return out[:n]


# speedup vs baseline: 53.5737x; 1.4354x over previous
import functools

import jax
import jax.numpy as jnp
from jax import lax
from jax.experimental import pallas as pl
from jax.experimental.pallas import tpu as pltpu
from jax.experimental.pallas import tpu_sc as plsc

_RANK = 12
_RES = 256
_OUT = 32
_L = 16
_B = 128
_NW = 32


def _sc_sample(np_pad, nblk):
  mesh = plsc.VectorSubcoreMesh(core_axis_name="c", subcore_axis_name="s")
  chunk = np_pad // _NW

  scratch = (
      [pltpu.VMEM((_B,), jnp.float32) for _ in range(6)]
      + [pltpu.VMEM((_B,), jnp.float32) for _ in range(6)]
      + [pltpu.VMEM((_B,), jnp.int32) for _ in range(24)]
      + [pltpu.VMEM((_B, _L), jnp.float32) for _ in range(24)]
      + [pltpu.VMEM((_B, 3 * _L), jnp.float32) for _ in range(2)]
      + [pltpu.SemaphoreType.DMA for _ in range(6)]
  )

  @functools.partial(
      pl.kernel,
      mesh=mesh,
      out_type=jax.ShapeDtypeStruct((np_pad, 3 * _L), jnp.float32),
      scratch_types=scratch,
      compiler_params=pltpu.CompilerParams(use_tc_tiling_on_sc=False),
  )
  def k(txy, txz, tyz, xs, ys, zs, f_out,
        cx0, cy0, cz0, cx1, cy1, cz1, w0a, w0b, w0c, w1a, w1b, w1c,
        *rest):
    i0 = rest[0:12]
    i1 = rest[12:24]
    g0 = rest[24:36]
    g1 = rest[36:48]
    fb0, fb1, cs0, cs1, gs0, gs1, fs0, fs1 = rest[48:56]
    wid = lax.axis_index("s") * 2 + lax.axis_index("c")
    base = wid * chunk
    tabs = (txy, txz, tyz)
    coords = (xs, ys, zs)
    cxyz = ((cx0, cy0, cz0), (cx1, cy1, cz1))
    wref = ((w0a, w0b, w0c), (w1a, w1b, w1c))
    idxs = (i0, i1)
    grows = (g0, g1)
    fbuf = (fb0, fb1)
    csem = (cs0, cs1)
    gsem = (gs0, gs1)
    fsem = (fs0, fs1)

    def coord_copies(k_blk, par):
      off = base + k_blk * _B
      return [
          pltpu.make_async_copy(coords[c].at[pl.ds(off, _B)], cxyz[par][c],
                                csem[par]) for c in range(3)
      ]

    def gather_copies(par):
      cps = []
      for pi in range(3):
        for t in range(4):
          j = 4 * pi + t
          cps.append(
              pltpu.make_async_copy(tabs[pi].at[idxs[par][j]], grows[par][j],
                                    gsem[par]))
      return cps

    def f_copy(k_blk, par):
      return pltpu.make_async_copy(
          fbuf[par], f_out.at[pl.ds(base + k_blk * _B, _B)], fsem[par])

    def idx_gen(par):
      for g in range(_B // _L):
        sl = pl.ds(g * _L, _L)

        def coord(c):
          w = cxyz[par][c][sl] * 255.0
          ui = jnp.clip(w.astype(jnp.int32), 0, 254)
          tu = jnp.clip(w - ui.astype(jnp.float32), 0.0, 1.0)
          return ui, tu

        xi, tx = coord(0)
        yi, ty = coord(1)
        zi, tz = coord(2)
        wref[par][0][sl] = tx
        wref[par][1][sl] = ty
        wref[par][2][sl] = tz
        for pi, (ci, ri) in enumerate(((xi, yi), (xi, zi), (yi, zi))):
          b0 = ri * 256 + ci
          idxs[par][4 * pi + 0][sl] = b0
          idxs[par][4 * pi + 1][sl] = b0 + 1
          idxs[par][4 * pi + 2][sl] = b0 + 256
          idxs[par][4 * pi + 3][sl] = b0 + 257

    def combine(par):
      def grp(g, c):
        b16 = g * _L
        txv = wref[par][0][pl.ds(b16, _L)]
        tyv = wref[par][1][pl.ds(b16, _L)]
        tzv = wref[par][2][pl.ds(b16, _L)]
        for j in range(_L):
          p = b16 + j
          tx = txv[j]
          ty = tyv[j]
          tz = tzv[j]
          for pi, (tc, tr) in enumerate(((tx, ty), (tx, tz), (ty, tz))):
            a = grows[par][4 * pi + 0][p]
            b = grows[par][4 * pi + 1][p]
            c2 = grows[par][4 * pi + 2][p]
            d = grows[par][4 * pi + 3][p]
            lo = a + tc * (b - a)
            hi = c2 + tc * (d - c2)
            fbuf[par][p, pl.ds(pi * _L, _L)] = lo + tr * (hi - lo)
        return c

      lax.fori_loop(0, _B // _L, grp, 0)

    def iteration(k_blk, issue_next_coord=True, wait_f=True):
      par = k_blk % 2 if isinstance(k_blk, int) else None
      assert par is not None
      for cp in coord_copies(k_blk, par):
        cp.wait()
      idx_gen(par)
      for cp in gather_copies(1 - par):
        cp.wait()
      for cp in gather_copies(par):
        cp.start()
      if issue_next_coord:
        for cp in coord_copies(k_blk + 1, 1 - par):
          cp.start()
      if wait_f:
        f_copy(k_blk - 3, 1 - par).wait()
      combine(1 - par)
      f_copy(k_blk - 1, 1 - par).start()

    def iteration_dyn(k_blk, par):
      for cp in coord_copies(k_blk, par):
        cp.wait()
      idx_gen(par)
      for cp in gather_copies(1 - par):
        cp.wait()
      for cp in gather_copies(par):
        cp.start()
      for cp in coord_copies(k_blk + 1, 1 - par):
        cp.start()
      f_copy(k_blk - 3, 1 - par).wait()
      combine(1 - par)
      f_copy(k_blk - 1, 1 - par).start()

    for c in range(3):
      pltpu.sync_copy(coords[c].at[pl.ds(base, _B)], cxyz[0][c])
    idx_gen(0)
    for cp in gather_copies(0):
      cp.start()
    for cp in coord_copies(1, 1):
      cp.start()
    iteration(1, wait_f=False)
    iteration(2, wait_f=False)

    def pair(m, c):
      k_blk = 3 + 2 * m
      iteration_dyn(k_blk, 1)
      iteration_dyn(k_blk + 1, 0)
      return c

    lax.fori_loop(0, (nblk - 4) // 2, pair, 0)

    iteration(nblk - 1, issue_next_coord=False)
    f_copy(nblk - 3, 1).wait()
    for cp in gather_copies(1):
      cp.wait()
    combine(1)
    f_copy(nblk - 1, 1).start()
    f_copy(nblk - 2, 0).wait()
    f_copy(nblk - 1, 1).wait()

  return k


def _tc_project(f, wp, n):
  blk = 4096
  grid = (n + blk - 1) // blk

  def mm(f_ref, w_ref, o_ref):
    o_ref[...] = jnp.dot(f_ref[...], w_ref[...],
                         preferred_element_type=jnp.float32)

  return pl.pallas_call(
      mm,
      grid=(grid,),
      in_specs=[
          pl.BlockSpec((blk, 3 * _L), lambda i: (i, 0)),
          pl.BlockSpec((3 * _L, _OUT), lambda i: (0, 0)),
      ],
      out_specs=pl.BlockSpec((blk, _OUT), lambda i: (i, 0)),
      out_shape=jax.ShapeDtypeStruct((n, _OUT), jnp.float32),
  )(f, wp)


def kernel(xyz, xy, xz, yz, W):
  n = xyz.shape[0]
  grain = 2 * _NW * _B
  np_pad = ((n + grain - 1) // grain) * grain
  nblk = (np_pad // _NW) // _B

  eye = jnp.pad(jnp.eye(_RANK, dtype=jnp.float32), ((0, 0), (0, _L - _RANK)))

  def tab(p):
    return jax.lax.dot_general(p.reshape(_RANK, _RES * _RES), eye,
                               (((0,), (0,)), ((), ())),
                               precision=jax.lax.Precision.HIGHEST)

  txy, txz, tyz = tab(xy), tab(xz), tab(yz)
  xyzp = jnp.pad(xyz, ((0, np_pad - n), (0, 0)))
  cs = jax.lax.dot_general(jnp.eye(3, dtype=jnp.float32), xyzp,
                           (((1,), (1,)), ((), ())),
                           precision=jax.lax.Precision.HIGHEST)

  f = _sc_sample(np_pad, nblk)(txy, txz, tyz, cs[0], cs[1], cs[2])

  wt = W.T
  wp = jnp.zeros((3 * _L, _OUT), jnp.float32)
  wp = (wp.at[0:_RANK].set(wt[0:_RANK])
          .at[_L:_L + _RANK].set(wt[_RANK:2 * _RANK])
          .at[2 * _L:2 * _L + _RANK].set(wt[2 * _RANK:3 * _RANK]))

  return _tc_project(f, wp, n)


# speedup vs baseline: 65.5030x; 1.2227x over previous
import functools

import jax
import jax.numpy as jnp
from jax import lax
from jax.experimental import pallas as pl
from jax.experimental.pallas import tpu as pltpu
from jax.experimental.pallas import tpu_sc as plsc

_RANK = 12
_RES = 256
_OUT = 32
_L = 16
_B = 128
_NW = 32


def _sc_sample(np_pad, nblk):
  mesh = plsc.VectorSubcoreMesh(core_axis_name="c", subcore_axis_name="s")
  chunk = np_pad // _NW

  scratch = (
      [pltpu.VMEM((_B,), jnp.float32) for _ in range(6)]
      + [pltpu.VMEM((_B,), jnp.float32) for _ in range(6)]
      + [pltpu.VMEM((_B,), jnp.int32) for _ in range(12)]
      + [pltpu.VMEM((_B, 2 * _L), jnp.float32) for _ in range(12)]
      + [pltpu.VMEM((_B, 8 * _L), jnp.float32) for _ in range(2)]
      + [pltpu.SemaphoreType.DMA for _ in range(6)]
  )

  @functools.partial(
      pl.kernel,
      mesh=mesh,
      out_type=jax.ShapeDtypeStruct((np_pad, 8 * _L), jnp.float32),
      scratch_types=scratch,
      compiler_params=pltpu.CompilerParams(use_tc_tiling_on_sc=False),
  )
  def k(txy, txz, tyz, xs, ys, zs, f_out,
        cx0, cy0, cz0, cx1, cy1, cz1, w0a, w0b, w0c, w1a, w1b, w1c,
        *rest):
    i0 = rest[0:6]
    i1 = rest[6:12]
    g0 = rest[12:18]
    g1 = rest[18:24]
    fb0, fb1, cs0, cs1, gs0, gs1, fs0, fs1 = rest[24:32]
    wid = lax.axis_index("s") * 2 + lax.axis_index("c")
    base = wid * chunk
    tabs = (txy, txz, tyz)
    coords = (xs, ys, zs)
    cxyz = ((cx0, cy0, cz0), (cx1, cy1, cz1))
    wref = ((w0a, w0b, w0c), (w1a, w1b, w1c))
    idxs = (i0, i1)
    grows = (g0, g1)
    fbuf = (fb0, fb1)
    csem = (cs0, cs1)
    gsem = (gs0, gs1)
    fsem = (fs0, fs1)

    def coord_copies(k_blk, par):
      off = base + k_blk * _B
      return [
          pltpu.make_async_copy(coords[c].at[pl.ds(off, _B)], cxyz[par][c],
                                csem[par]) for c in range(3)
      ]

    def gather_copies(par):
      cps = []
      for pi in range(3):
        for t in range(2):
          j = 2 * pi + t
          cps.append(
              pltpu.make_async_copy(tabs[pi].at[idxs[par][j]], grows[par][j],
                                    gsem[par]))
      return cps

    def f_copy(k_blk, par):
      return pltpu.make_async_copy(
          fbuf[par], f_out.at[pl.ds(base + k_blk * _B, _B)], fsem[par])

    def idx_gen(par):
      for g in range(_B // _L):
        sl = pl.ds(g * _L, _L)

        def coord(c):
          w = cxyz[par][c][sl] * 255.0
          ui = jnp.clip(w.astype(jnp.int32), 0, 254)
          tu = jnp.clip(w - ui.astype(jnp.float32), 0.0, 1.0)
          return ui, tu

        xi, tx = coord(0)
        yi, ty = coord(1)
        zi, tz = coord(2)
        wref[par][0][sl] = tx
        wref[par][1][sl] = ty
        wref[par][2][sl] = tz
        for pi, (ci, ri) in enumerate(((xi, yi), (xi, zi), (yi, zi))):
          b0 = ri * 256 + ci
          idxs[par][2 * pi + 0][sl] = b0
          idxs[par][2 * pi + 1][sl] = b0 + 256

    def combine(par):
      def grp(g, c):
        b16 = g * _L
        txv = wref[par][0][pl.ds(b16, _L)]
        tyv = wref[par][1][pl.ds(b16, _L)]
        tzv = wref[par][2][pl.ds(b16, _L)]
        for j in range(_L):
          p = b16 + j
          tx = txv[j]
          ty = tyv[j]
          tz = tzv[j]
          for pi, (tc, tr) in enumerate(((tx, ty), (tx, tz), (ty, tz))):
            r0 = grows[par][2 * pi + 0]
            r1 = grows[par][2 * pi + 1]
            a = r0[p, pl.ds(0, _L)]
            b = r0[p, pl.ds(_L, _L)]
            c2 = r1[p, pl.ds(0, _L)]
            d = r1[p, pl.ds(_L, _L)]
            lo = a + tc * (b - a)
            hi = c2 + tc * (d - c2)
            fbuf[par][p, pl.ds(pi * _L, _L)] = lo + tr * (hi - lo)
        return c

      lax.fori_loop(0, _B // _L, grp, 0)

    def iteration(k_blk, issue_next_coord=True, wait_f=True):
      par = k_blk % 2 if isinstance(k_blk, int) else None
      assert par is not None
      for cp in coord_copies(k_blk, par):
        cp.wait()
      idx_gen(par)
      for cp in gather_copies(1 - par):
        cp.wait()
      for cp in gather_copies(par):
        cp.start()
      if issue_next_coord:
        for cp in coord_copies(k_blk + 1, 1 - par):
          cp.start()
      if wait_f:
        f_copy(k_blk - 3, 1 - par).wait()
      combine(1 - par)
      f_copy(k_blk - 1, 1 - par).start()

    def iteration_dyn(k_blk, par):
      for cp in coord_copies(k_blk, par):
        cp.wait()
      idx_gen(par)
      for cp in gather_copies(1 - par):
        cp.wait()
      for cp in gather_copies(par):
        cp.start()
      for cp in coord_copies(k_blk + 1, 1 - par):
        cp.start()
      f_copy(k_blk - 3, 1 - par).wait()
      combine(1 - par)
      f_copy(k_blk - 1, 1 - par).start()

    zv = jnp.zeros((_L,), jnp.float32)

    def zrow(r, c):
      for par in range(2):
        for l in range(3, 8):
          fbuf[par][r, pl.ds(l * _L, _L)] = zv
      return c

    lax.fori_loop(0, _B, zrow, 0)

    for c in range(3):
      pltpu.sync_copy(coords[c].at[pl.ds(base, _B)], cxyz[0][c])
    idx_gen(0)
    for cp in gather_copies(0):
      cp.start()
    for cp in coord_copies(1, 1):
      cp.start()
    iteration(1, wait_f=False)
    iteration(2, wait_f=False)

    def pair(m, c):
      k_blk = 3 + 2 * m
      iteration_dyn(k_blk, 1)
      iteration_dyn(k_blk + 1, 0)
      return c

    lax.fori_loop(0, (nblk - 4) // 2, pair, 0)

    iteration(nblk - 1, issue_next_coord=False)
    f_copy(nblk - 3, 1).wait()
    for cp in gather_copies(1):
      cp.wait()
    combine(1)
    f_copy(nblk - 1, 1).start()
    f_copy(nblk - 2, 0).wait()
    f_copy(nblk - 1, 1).wait()

  return k


def _tc_project(f, wp, n):
  blk = 4096
  grid = (n + blk - 1) // blk

  def mm(f_ref, w_ref, o_ref):
    o_ref[...] = jnp.dot(f_ref[...], w_ref[...],
                         preferred_element_type=jnp.float32)

  return pl.pallas_call(
      mm,
      grid=(grid,),
      in_specs=[
          pl.BlockSpec((blk, 8 * _L), lambda i: (i, 0)),
          pl.BlockSpec((8 * _L, _OUT), lambda i: (0, 0)),
      ],
      out_specs=pl.BlockSpec((blk, _OUT), lambda i: (i, 0)),
      out_shape=jax.ShapeDtypeStruct((n, _OUT), jnp.float32),
  )(f, wp)


def kernel(xyz, xy, xz, yz, W):
  n = xyz.shape[0]
  grain = 2 * _NW * _B
  np_pad = ((n + grain - 1) // grain) * grain
  nblk = (np_pad // _NW) // _B

  eye = jnp.pad(jnp.eye(_RANK, dtype=jnp.float32), ((0, 0), (0, _L - _RANK)))

  def tab(p):
    t = jax.lax.dot_general(p.reshape(_RANK, _RES * _RES), eye,
                            (((0,), (0,)), ((), ())),
                            precision=jax.lax.Precision.HIGHEST)
    tn = jnp.concatenate([t[1:], t[-1:]], axis=0)
    return jnp.concatenate([t, tn], axis=1)

  txy, txz, tyz = tab(xy), tab(xz), tab(yz)
  xyzp = jnp.pad(xyz, ((0, np_pad - n), (0, 0)))
  cs = jax.lax.dot_general(jnp.eye(3, dtype=jnp.float32), xyzp,
                           (((1,), (1,)), ((), ())),
                           precision=jax.lax.Precision.HIGHEST)

  f = _sc_sample(np_pad, nblk)(txy, txz, tyz, cs[0], cs[1], cs[2])

  wt = W.T
  wp = jnp.zeros((8 * _L, _OUT), jnp.float32)
  wp = (wp.at[0:_RANK].set(wt[0:_RANK])
          .at[_L:_L + _RANK].set(wt[_RANK:2 * _RANK])
          .at[2 * _L:2 * _L + _RANK].set(wt[2 * _RANK:3 * _RANK]))

  return _tc_project(f, wp, n)
